# baseline (speedup 1.0000x reference)
import functools

import jax
import jax.numpy as jnp
from jax import lax
from jax.experimental import pallas as pl
from jax.experimental.pallas import tpu as pltpu
from jax.experimental.pallas import tpu_sc as plsc

NC = 2
NS = 16
NW = NC * NS
LANES = 16


@functools.partial(jax.jit, static_argnums=(3, 4, 5))
def _sc_embed(ids, token_table, position_table, B, L, E):
    seq_per_w = B // NW
    scale = 8.0

    mesh = plsc.VectorSubcoreMesh(core_axis_name="c", subcore_axis_name="s")

    @functools.partial(
        pl.kernel,
        out_type=jax.ShapeDtypeStruct((B * L, E), jnp.float32),
        mesh=mesh,
        scratch_types=[
            pltpu.VMEM((L,), jnp.int32),
            pltpu.VMEM((L, E), jnp.float32),
            pltpu.VMEM((L, E), jnp.float32),
            pltpu.VMEM((L, E), jnp.float32),
            pltpu.SemaphoreType.DMA,
        ],
        compiler_params=pltpu.CompilerParams(use_tc_tiling_on_sc=False),
    )
    def k(table_hbm, ids_hbm, pos_hbm, out_hbm, ids_v, rows_v, out_v, pos_v, sem):
        wid = lax.axis_index("s") * NC + lax.axis_index("c")

        pltpu.sync_copy(pos_hbm, pos_v)

        @pl.loop(0, seq_per_w)
        def _(s):
            seq = wid * seq_per_w + s
            base = seq * L
            pltpu.sync_copy(ids_hbm.at[pl.ds(base, L)], ids_v)
            c1 = pltpu.async_copy(
                table_hbm.at[ids_v.at[pl.ds(0, 128)]], rows_v.at[pl.ds(0, 128)], sem
            )
            c2 = pltpu.async_copy(
                table_hbm.at[ids_v.at[pl.ds(128, L - 128)]],
                rows_v.at[pl.ds(128, L - 128)],
                sem,
            )
            c1.wait()
            c2.wait()

            def do_rows(base16, j_lo, j_hi):
                idvec = ids_v[pl.ds(base16, LANES)]
                mvec = jnp.where(idvec == 0, 0.0, 1.0)
                for j in range(j_lo, j_hi):
                    m = mvec[j]
                    w = base16 + j
                    for c in range(E // LANES):
                        sl = pl.ds(c * LANES, LANES)
                        out_v[w, sl] = (rows_v[w, sl] * scale + pos_v[w, sl]) * m

            @pl.loop(0, L // LANES)
            def _(g):
                do_rows(g * LANES, 0, LANES)

            if L % LANES:
                do_rows(L - LANES, LANES - L % LANES, LANES)

            pltpu.sync_copy(out_v, out_hbm.at[pl.ds(base, L)])

    return k(token_table, ids, position_table)


def kernel(inputs, token_table, position_table):
    B, L = inputs.shape
    V, E = token_table.shape
    ids = inputs.reshape(-1).astype(jnp.int32)
    out = _sc_embed(ids, token_table, position_table, B, L, E)
    return out.reshape(B, L, E)
